# Initial kernel scaffold; baseline (speedup 1.0000x reference)
import jax
import jax.numpy as jnp
from jax.experimental import pallas as pl


def kernel(node_feats, edge_attrs, tp_weights, sender_list, receiver_list, first_occurences):
    raise NotImplementedError("write your pallas kernel here")


# speedup vs baseline: 5.7335x; 5.7335x over previous
import functools

import jax
import jax.numpy as jnp
from jax import lax
from jax.experimental import pallas as pl
from jax.experimental.pallas import tpu as pltpu
from jax.experimental.pallas import tpu_sc as plsc

L_IDX = (0, 1, 1, 1, 2, 2, 2, 2, 2, 3, 3, 3, 3, 3, 3, 3)
LANES = 16
NC, NS = 2, 16
NW = NC * NS
EB = 128


def _zero_ref(ref, rows, cols):
    z = jnp.zeros((LANES,), jnp.float32)
    for r in range(rows):
        for c in range(cols // LANES):
            ref[r, pl.ds(c * LANES, LANES)] = z


def _sc_body(node_feats, edge_attrs, tp_w, sender, receiver, params, out,
             p_v, sidx_v, r_v, a_v, s_v, w_v, acc, zbuf, sem):
    nfc = 128 // LANES

    wid = lax.axis_index("s") * NC + lax.axis_index("c")
    pltpu.sync_copy(params.at[wid], p_v)
    p_vec = p_v[:]
    e_start = p_vec[0]
    e_end = p_vec[1]
    r_start = p_vec[2]
    r_end = p_vec[3]

    _zero_ref(acc, 16, 128)
    _zero_ref(zbuf, 16, 128)

    def fill_rows(lo, hi):
        def f(rr, c):
            pltpu.sync_copy(zbuf, out.at[rr])
            return c
        lax.fori_loop(lo, hi, f, 0)

    def accumulate(i):
        q = []
        for l in range(4):
            ql = []
            for fc in range(nfc):
                s_fc = s_v[i, pl.ds(fc * LANES, LANES)]
                ql.append(s_fc * w_v[i, l, pl.ds(fc * LANES, LANES)])
            q.append(ql)
        a_row = a_v[i, :]
        for m in range(16):
            a_bcast = jnp.full((LANES,), a_row[m], jnp.float32)
            ql = q[L_IDX[m]]
            for fc in range(nfc):
                plsc.addupdate(acc.at[m, pl.ds(fc * LANES, LANES)],
                               a_bcast * ql[fc])

    def edge_body(i, r_cur):
        r_e = r_v[pl.ds(i, LANES)][0]

        @pl.when(r_e != r_cur)
        def _flush():
            pltpu.sync_copy(acc, out.at[r_cur])
            _zero_ref(acc, 16, 128)
            fill_rows(r_cur + 1, r_e)

        accumulate(i)
        return r_e

    def block_body(b, r_cur):
        eb = b * EB
        pltpu.sync_copy(sender.at[pl.ds(eb, EB)], sidx_v)
        pltpu.sync_copy(receiver.at[pl.ds(eb, EB)], r_v.at[pl.ds(0, EB)])
        pltpu.sync_copy(edge_attrs.at[pl.ds(eb, EB)], a_v)
        pltpu.sync_copy(tp_w.at[pl.ds(eb, EB)], w_v)
        pltpu.async_copy(node_feats.at[sidx_v], s_v, sem).wait()
        lo_i = jnp.maximum(e_start - eb, 0)
        hi_i = jnp.minimum(e_end - eb, EB)
        return lax.fori_loop(lo_i, hi_i, edge_body, r_cur)

    b_lo = e_start // EB
    b_hi = (e_end + EB - 1) // EB
    r_cur = lax.fori_loop(b_lo, b_hi, block_body, r_start)

    @pl.when(r_end > r_start)
    def _final():
        pltpu.sync_copy(acc, out.at[r_cur])
        fill_rows(r_cur + 1, r_end)


def kernel(node_feats, edge_attrs, tp_weights, sender_list, receiver_list,
           first_occurences):
    n, f = node_feats.shape
    e = edge_attrs.shape[0]

    fo_ext = jnp.concatenate(
        [first_occurences.astype(jnp.int32),
         jnp.array([e], jnp.int32)])
    targets = (jnp.arange(NW, dtype=jnp.int32) * (e // NW)).astype(jnp.int32)
    b = jnp.searchsorted(fo_ext, targets, side="left").astype(jnp.int32)
    b_ext = jnp.concatenate([b, jnp.array([n], jnp.int32)])
    e_starts = fo_ext[b_ext[:-1]]
    e_ends = fo_ext[b_ext[1:]]
    params = jnp.zeros((NW, 16), jnp.int32)
    params = (params.at[:, 0].set(e_starts)
                    .at[:, 1].set(e_ends)
                    .at[:, 2].set(b_ext[:-1])
                    .at[:, 3].set(b_ext[1:]))

    mesh = plsc.VectorSubcoreMesh(core_axis_name="c", subcore_axis_name="s",
                                  num_cores=NC, num_subcores=NS)
    run = functools.partial(
        pl.kernel,
        out_type=jax.ShapeDtypeStruct((n, 16, f), jnp.float32),
        mesh=mesh,
        scratch_types=[
            pltpu.VMEM((LANES,), jnp.int32),
            pltpu.VMEM((EB,), jnp.int32),
            pltpu.VMEM((EB + LANES,), jnp.int32),
            pltpu.VMEM((EB, 16), jnp.float32),
            pltpu.VMEM((EB, f), jnp.float32),
            pltpu.VMEM((EB, 4, f), jnp.float32),
            pltpu.VMEM((16, f), jnp.float32),
            pltpu.VMEM((16, f), jnp.float32),
            pltpu.SemaphoreType.DMA,
        ],
    )(_sc_body)
    return run(node_feats, edge_attrs, tp_weights, sender_list.astype(jnp.int32),
               receiver_list.astype(jnp.int32), params)
